# Initial kernel scaffold; baseline (speedup 1.0000x reference)
import jax
import jax.numpy as jnp
from jax.experimental import pallas as pl


def kernel(points, W1, b1, g1, be1, W2, b2, g2, be2):
    raise NotImplementedError("write your pallas kernel here")


# speedup vs baseline: 7.2748x; 7.2748x over previous
import functools

import jax
import jax.numpy as jnp
from jax import lax
from jax.experimental import pallas as pl
from jax.experimental.pallas import tpu as pltpu

GRID_X = 256
GRID_Y = 256
NCELL = GRID_X * GRID_Y
MAX_VOXELS = 12000
MAX_PTS = 32
VB = 1000


def _voxelize(points):
    B, N, _ = points.shape
    sent = NCELL

    def one(pts):
        m = jnp.all((pts[:, :3] >= 0.0) & (pts[:, :3] < 1.0), axis=1)
        vx = jnp.clip((pts[:, 0] * GRID_X).astype(jnp.int32), 0, GRID_X - 1)
        vy = jnp.clip((pts[:, 1] * GRID_Y).astype(jnp.int32), 0, GRID_Y - 1)
        h = vx * GRID_Y + vy
        hs = jnp.where(m, h, sent)
        pos = jnp.arange(N, dtype=jnp.int32)
        sh, order = lax.sort((hs, pos), num_keys=1, is_stable=True)
        is_new = jnp.concatenate([jnp.array([True]), sh[1:] != sh[:-1]])
        sinv = jnp.cumsum(is_new.astype(jnp.int32)) - 1
        first = lax.cummax(jnp.where(is_new, pos, -1))
        within = pos - first
        keep = (sh < sent) & (within < MAX_PTS) & (sinv < MAX_VOXELS)
        si = jnp.where(keep, sinv, MAX_VOXELS)
        wi = jnp.where(keep, within, MAX_PTS)
        psorted = pts[order]
        vox = jnp.zeros((MAX_VOXELS, MAX_PTS, 4), jnp.float32).at[si, wi].set(
            psorted, mode='drop')
        val = jnp.zeros((MAX_VOXELS, MAX_PTS), jnp.float32).at[si, wi].set(
            1.0, mode='drop')
        newu = is_new & (sh < sent) & (sinv < MAX_VOXELS)
        ui = jnp.where(newu, sinv, MAX_VOXELS)
        uh = jnp.zeros((MAX_VOXELS,), jnp.int32).at[ui].set(
            sh.astype(jnp.int32), mode='drop')
        return vox, val, uh

    return jax.vmap(one)(points)


def _pass1_body(vox_ref, val_ref, w1a_ref, w1b_ref, out_ref, acc_ref):
    vox = vox_ref[...]
    val = val_ref[...]
    pcnt = jnp.maximum(jnp.sum(val, axis=1), 1.0)
    center = jnp.sum(vox, axis=1) / pcnt[:, None]
    off = vox[..., :3] - center[:, None, :3]
    R = VB * MAX_PTS
    z = (jnp.dot(vox.reshape(R, 4), w1a_ref[...],
                 preferred_element_type=jnp.float32)
         + jnp.dot(off.reshape(R, 3), w1b_ref[...],
                   preferred_element_type=jnp.float32))
    s = jnp.sum(z, axis=0)
    s2 = jnp.sum(z * z, axis=0)

    @pl.when(pl.program_id(0) == 0)
    def _():
        acc_ref[...] = jnp.zeros_like(acc_ref)

    acc_ref[0, :] += s
    acc_ref[1, :] += s2

    @pl.when(pl.program_id(0) == pl.num_programs(0) - 1)
    def _():
        out_ref[...] = acc_ref[...]


def _pass2_body(vox_ref, val_ref, w1a_ref, w1b_ref, a1_ref, c1_ref, w2_ref,
                out_ref, acc_ref):
    vox = vox_ref[...]
    val = val_ref[...]
    pcnt = jnp.maximum(jnp.sum(val, axis=1), 1.0)
    center = jnp.sum(vox, axis=1) / pcnt[:, None]
    off = vox[..., :3] - center[:, None, :3]
    R = VB * MAX_PTS
    z = (jnp.dot(vox.reshape(R, 4), w1a_ref[...],
                 preferred_element_type=jnp.float32)
         + jnp.dot(off.reshape(R, 3), w1b_ref[...],
                   preferred_element_type=jnp.float32))
    h1 = jnp.maximum(z * a1_ref[...] + c1_ref[...], 0.0)
    z2 = jnp.dot(h1, w2_ref[...], preferred_element_type=jnp.float32)
    s = jnp.sum(z2, axis=0)
    s2 = jnp.sum(z2 * z2, axis=0)

    @pl.when(pl.program_id(0) == 0)
    def _():
        acc_ref[...] = jnp.zeros_like(acc_ref)

    acc_ref[0, :] += s
    acc_ref[1, :] += s2

    @pl.when(pl.program_id(0) == pl.num_programs(0) - 1)
    def _():
        out_ref[...] = acc_ref[...]


def _pass3_body(vox_ref, val_ref, w1a_ref, w1b_ref, a1_ref, c1_ref, w2_ref,
                a2_ref, c2_ref, out_ref):
    vox = vox_ref[...]
    val = val_ref[...]
    pcnt = jnp.maximum(jnp.sum(val, axis=1), 1.0)
    center = jnp.sum(vox, axis=1) / pcnt[:, None]
    off = vox[..., :3] - center[:, None, :3]
    R = VB * MAX_PTS
    z = (jnp.dot(vox.reshape(R, 4), w1a_ref[...],
                 preferred_element_type=jnp.float32)
         + jnp.dot(off.reshape(R, 3), w1b_ref[...],
                   preferred_element_type=jnp.float32))
    h1 = jnp.maximum(z * a1_ref[...] + c1_ref[...], 0.0)
    z2 = jnp.dot(h1, w2_ref[...], preferred_element_type=jnp.float32)
    h2 = jnp.maximum(z2 * a2_ref[...] + c2_ref[...], 0.0)
    h2 = h2.reshape(VB, MAX_PTS, 64) * val[..., None]
    out_ref[...] = jnp.max(h2, axis=1)


def _mlp_pipeline(vox, val, W1, b1, g1, be1, W2, b2, g2, be2):
    BV = vox.shape[0]
    nblk = BV // VB
    Ntot = jnp.float32(BV * MAX_PTS)
    w1a = W1[0:4, :]
    w1b = W1[4:7, :] + W1[7:10, :]

    grid = (nblk,)
    vox_spec = pl.BlockSpec((VB, MAX_PTS, 4), lambda i: (i, 0, 0))
    val_spec = pl.BlockSpec((VB, MAX_PTS), lambda i: (i, 0))
    w1a_spec = pl.BlockSpec((4, 64), lambda i: (0, 0))
    w1b_spec = pl.BlockSpec((3, 64), lambda i: (0, 0))
    vec_spec = pl.BlockSpec((1, 64), lambda i: (0, 0))
    w2_spec = pl.BlockSpec((64, 64), lambda i: (0, 0))
    stat_spec = pl.BlockSpec((2, 64), lambda i: (0, 0))

    stats1 = pl.pallas_call(
        _pass1_body,
        grid=grid,
        in_specs=[vox_spec, val_spec, w1a_spec, w1b_spec],
        out_specs=stat_spec,
        out_shape=jax.ShapeDtypeStruct((2, 64), jnp.float32),
        scratch_shapes=[pltpu.VMEM((2, 64), jnp.float32)],
    )(vox, val, w1a, w1b)

    mz1 = stats1[0] / Ntot
    var1 = stats1[1] / Ntot - mz1 * mz1
    sc1 = g1 / jnp.sqrt(var1 + 1e-5)
    a1 = sc1[None, :]
    c1 = (be1 - mz1 * sc1)[None, :]

    stats2 = pl.pallas_call(
        _pass2_body,
        grid=grid,
        in_specs=[vox_spec, val_spec, w1a_spec, w1b_spec, vec_spec, vec_spec,
                  w2_spec],
        out_specs=stat_spec,
        out_shape=jax.ShapeDtypeStruct((2, 64), jnp.float32),
        scratch_shapes=[pltpu.VMEM((2, 64), jnp.float32)],
    )(vox, val, w1a, w1b, a1, c1, W2)

    mz2 = stats2[0] / Ntot
    var2 = stats2[1] / Ntot - mz2 * mz2
    sc2 = g2 / jnp.sqrt(var2 + 1e-5)
    a2 = sc2[None, :]
    c2 = (be2 - mz2 * sc2)[None, :]

    feats = pl.pallas_call(
        _pass3_body,
        grid=grid,
        in_specs=[vox_spec, val_spec, w1a_spec, w1b_spec, vec_spec, vec_spec,
                  w2_spec, vec_spec, vec_spec],
        out_specs=pl.BlockSpec((VB, 64), lambda i: (i, 0)),
        out_shape=jax.ShapeDtypeStruct((BV, 64), jnp.float32),
    )(vox, val, w1a, w1b, a1, c1, W2, a2, c2)
    return feats


def kernel(points, W1, b1, g1, be1, W2, b2, g2, be2):
    B = points.shape[0]
    del b1, b2

    vox, val, uh = _voxelize(points)
    vox = vox.reshape(B * MAX_VOXELS, MAX_PTS, 4)
    val = val.reshape(B * MAX_VOXELS, MAX_PTS)

    feats = _mlp_pipeline(vox, val, W1, None, g1, be1, W2, None, g2, be2)
    feats = feats.reshape(B, MAX_VOXELS, 64)

    x = uh // GRID_Y
    y = uh % GRID_Y
    bev = jnp.zeros((B, GRID_Y, GRID_X, 64), jnp.float32)
    bev = bev.at[jnp.arange(B)[:, None], y, x].set(feats)
    return jnp.transpose(bev, (0, 3, 1, 2))
